# Initial kernel scaffold; baseline (speedup 1.0000x reference)
import functools

import jax
import jax.numpy as jnp
from jax import lax
from jax.experimental import pallas as pl
from jax.experimental.pallas import tpu as pltpu
from jax.experimental.pallas import tpu_sc as plsc

N = 10000
E = 320000
D_IN = 128
D_EDGE = 16
D_OUT = 128

NC = 2
NS = 16
NW = NC * NS
LANES = 16

EW = E // NW
CH = 80
NCHUNK = EW // CH
RPT = N // NS
ZR = 125

BN = 1000
BE = 4000


def _h_body(nf_ref, wn_ref, bn_ref, h_ref):
    h_ref[...] = (
        lax.dot_general(
            nf_ref[...], wn_ref[...],
            (((1,), (1,)), ((), ())),
            preferred_element_type=jnp.float32,
        )
        + bn_ref[...]
    )


def _ef_body(ef_in_ref, we_ref, be_ref, ef_ref):
    ef_ref[...] = (
        lax.dot_general(
            ef_in_ref[...], we_ref[...],
            (((1,), (1,)), ((), ())),
            preferred_element_type=jnp.float32,
        )
        + be_ref[...]
    )


def _combine_body(p_ref, h_ref, resw_ref, degs_ref, o_ref):
    h = h_ref[...]
    res = jnp.maximum(h + resw_ref[...], 0.0) * (1.0 / degs_ref[...])
    o_ref[...] = p_ref[0] + p_ref[1] + res


def _sc_edge_body(src_hbm, dst_hbm, norm_hbm, ef_hbm, h_hbm, out_hbm,
                  src_v, dst_v, rows_v, ef_v, norm_v, zbuf, agg, sem):
    cid = lax.axis_index("c")
    sid = lax.axis_index("s")
    wid = cid * NS + sid

    def _zero_row(r, carry):
        for c8 in range(D_OUT // LANES):
            zbuf[r, pl.ds(c8 * LANES, LANES)] = jnp.zeros((LANES,), jnp.float32)
        return carry

    lax.fori_loop(0, ZR, _zero_row, 0)
    r0 = sid * RPT
    for k in range(RPT // ZR):
        pltpu.sync_copy(zbuf, agg.at[pl.ds(r0 + k * ZR, ZR), :])
    plsc.subcore_barrier()

    ebase = wid * EW

    def _chunk(ci, carry):
        off = ebase + ci * CH
        pltpu.sync_copy(src_hbm.at[pl.ds(off, CH)], src_v)
        pltpu.sync_copy(dst_hbm.at[pl.ds(off, CH)], dst_v)
        pltpu.sync_copy(norm_hbm.at[pl.ds(off, CH)], norm_v)
        pltpu.async_copy(h_hbm.at[src_v], rows_v, sem).wait()
        pltpu.sync_copy(ef_hbm.at[pl.ds(off, CH), :], ef_v)

        def _edge(e, c2):
            nsp = plsc.load_gather(norm_v, [lax.broadcast(e, (LANES,))])
            for v in range(D_OUT // LANES):
                x = rows_v[e, pl.ds(v * LANES, LANES)]
                y = ef_v[e, pl.ds(v * LANES, LANES)]
                rows_v[e, pl.ds(v * LANES, LANES)] = (
                    jnp.maximum(x + y, 0.0) * nsp
                )
            return c2

        lax.fori_loop(0, CH, _edge, 0)
        pltpu.sync_copy(rows_v, agg.at[dst_v], add=True)
        return carry

    lax.fori_loop(0, NCHUNK, _chunk, 0)
    plsc.subcore_barrier()

    for k in range(RPT // ZR):
        rr = r0 + k * ZR
        pltpu.sync_copy(agg.at[pl.ds(rr, ZR), :], out_hbm.at[cid, pl.ds(rr, ZR), :])


_sc_edge = functools.partial(
    pl.kernel,
    out_type=jax.ShapeDtypeStruct((NC, N, D_OUT), jnp.float32),
    mesh=plsc.VectorSubcoreMesh(core_axis_name="c", subcore_axis_name="s"),
    scratch_types=[
        pltpu.VMEM((CH,), jnp.int32),
        pltpu.VMEM((CH,), jnp.int32),
        pltpu.VMEM((CH, D_OUT), jnp.float32),
        pltpu.VMEM((CH, D_OUT), jnp.float32),
        pltpu.VMEM((CH,), jnp.float32),
        pltpu.VMEM((ZR, D_OUT), jnp.float32),
        pltpu.VMEM_SHARED((N, D_OUT), jnp.float32),
        pltpu.SemaphoreType.DMA,
    ],
)(_sc_edge_body)


@jax.jit
def kernel(node_feats, edge_feats, degs, norm, Wn, bn, We, be, res_w, edge_index):
    src = edge_index[0]
    dst = edge_index[1]
    norm_flat = norm.reshape(E)
    bn2 = bn.reshape(1, D_OUT)
    be2 = be.reshape(1, D_OUT)

    h = pl.pallas_call(
        _h_body,
        grid=(N // BN,),
        in_specs=[
            pl.BlockSpec((BN, D_IN), lambda i: (i, 0)),
            pl.BlockSpec((D_OUT, D_IN), lambda i: (0, 0)),
            pl.BlockSpec((1, D_OUT), lambda i: (0, 0)),
        ],
        out_specs=pl.BlockSpec((BN, D_OUT), lambda i: (i, 0)),
        out_shape=jax.ShapeDtypeStruct((N, D_OUT), jnp.float32),
    )(node_feats, Wn, bn2)

    ef = pl.pallas_call(
        _ef_body,
        grid=(E // BE,),
        in_specs=[
            pl.BlockSpec((BE, D_EDGE), lambda i: (i, 0)),
            pl.BlockSpec((D_OUT, D_EDGE), lambda i: (0, 0)),
            pl.BlockSpec((1, D_OUT), lambda i: (0, 0)),
        ],
        out_specs=pl.BlockSpec((BE, D_OUT), lambda i: (i, 0)),
        out_shape=jax.ShapeDtypeStruct((E, D_OUT), jnp.float32),
    )(edge_feats, We, be2)

    partials = _sc_edge(src, dst, norm_flat, ef, h)

    out = pl.pallas_call(
        _combine_body,
        grid=(N // BN,),
        in_specs=[
            pl.BlockSpec((NC, BN, D_OUT), lambda i: (0, i, 0)),
            pl.BlockSpec((BN, D_OUT), lambda i: (i, 0)),
            pl.BlockSpec((1, D_OUT), lambda i: (0, 0)),
            pl.BlockSpec((BN, 1), lambda i: (i, 0)),
        ],
        out_specs=pl.BlockSpec((BN, D_OUT), lambda i: (i, 0)),
        out_shape=jax.ShapeDtypeStruct((N, D_OUT), jnp.float32),
    )(partials, h, res_w, degs)

    return out


# speedup vs baseline: 2.4127x; 2.4127x over previous
import functools

import jax
import jax.numpy as jnp
from jax import lax
from jax.experimental import pallas as pl
from jax.experimental.pallas import tpu as pltpu
from jax.experimental.pallas import tpu_sc as plsc

N = 10000
E = 320000
D_IN = 128
D_EDGE = 16
D_OUT = 128

NC = 2
NS = 16
NW = NC * NS
LANES = 16

EW = E // NW
CH = 80
NCHUNK = EW // CH
ROWS_MAIN = 640
ROWS_LAST = N - (NS - 1) * ROWS_MAIN
ZR = 128

BN = 1000
BE = 4000


def _h_body(nf_ref, wn_ref, bn_ref, h_ref):
    h_ref[...] = (
        lax.dot_general(
            nf_ref[...], wn_ref[...],
            (((1,), (1,)), ((), ())),
            preferred_element_type=jnp.float32,
        )
        + bn_ref[...]
    )


def _ef_body(ef_in_ref, we_ref, be_ref, ef_ref):
    ef_ref[...] = (
        lax.dot_general(
            ef_in_ref[...], we_ref[...],
            (((1,), (1,)), ((), ())),
            preferred_element_type=jnp.float32,
        )
        + be_ref[...]
    )


def _combine_body(p_ref, h_ref, resw_ref, degs_ref, o_ref):
    h = h_ref[...]
    res = jnp.maximum(h + resw_ref[...], 0.0) * (1.0 / degs_ref[...])
    o_ref[...] = p_ref[0] + p_ref[1] + res


def _sc_edge_body(src_hbm, dst_hbm, norm_hbm, ef_hbm, h_hbm, out_hbm,
                  src_v, dst_v, rows_v, ef_v, norm_v, zbuf, agg, sem):
    cid = lax.axis_index("c")
    sid = lax.axis_index("s")
    wid = cid * NS + sid

    def _zero_row(r, carry):
        for c8 in range(D_OUT // LANES):
            zbuf[r, pl.ds(c8 * LANES, LANES)] = jnp.zeros((LANES,), jnp.float32)
        return carry

    lax.fori_loop(0, ZR, _zero_row, 0)
    r0 = pl.multiple_of(sid * ROWS_MAIN, ROWS_MAIN)

    @pl.when(sid < NS - 1)
    def _zero_main():
        for k in range(ROWS_MAIN // ZR):
            pltpu.sync_copy(zbuf, agg.at[pl.ds(r0 + k * ZR, ZR), :])

    @pl.when(sid == NS - 1)
    def _zero_last():
        base = (NS - 1) * ROWS_MAIN
        for k in range(ROWS_LAST // ZR):
            pltpu.sync_copy(zbuf, agg.at[pl.ds(base + k * ZR, ZR), :])
        rem = ROWS_LAST % ZR
        if rem:
            pltpu.sync_copy(
                zbuf.at[pl.ds(0, rem), :],
                agg.at[pl.ds(base + (ROWS_LAST // ZR) * ZR, rem), :],
            )

    plsc.subcore_barrier()

    ebase = wid * EW

    def _chunk(ci, carry):
        off = ebase + ci * CH
        pltpu.sync_copy(src_hbm.at[pl.ds(off, CH)], src_v)
        pltpu.sync_copy(dst_hbm.at[pl.ds(off, CH)], dst_v)
        pltpu.sync_copy(norm_hbm.at[pl.ds(off, CH)], norm_v.at[pl.ds(0, CH)])
        pltpu.async_copy(h_hbm.at[src_v], rows_v, sem).wait()
        pltpu.sync_copy(ef_hbm.at[pl.ds(off, CH), :], ef_v)

        def _edge(e, c2):
            nv = norm_v[pl.ds(e, LANES)]
            nsp = lax.broadcast(nv[0], (LANES,))
            for v in range(D_OUT // LANES):
                x = rows_v[e, pl.ds(v * LANES, LANES)]
                y = ef_v[e, pl.ds(v * LANES, LANES)]
                rows_v[e, pl.ds(v * LANES, LANES)] = (
                    jnp.maximum(x + y, 0.0) * nsp
                )
            return c2

        lax.fori_loop(0, CH, _edge, 0)
        pltpu.sync_copy(rows_v, agg.at[dst_v], add=True)
        return carry

    lax.fori_loop(0, NCHUNK, _chunk, 0)
    plsc.subcore_barrier()

    @pl.when(sid < NS - 1)
    def _pub_main():
        pltpu.sync_copy(
            agg.at[pl.ds(r0, ROWS_MAIN), :],
            out_hbm.at[cid, pl.ds(r0, ROWS_MAIN), :],
        )

    @pl.when(sid == NS - 1)
    def _pub_last():
        base = (NS - 1) * ROWS_MAIN
        pltpu.sync_copy(
            agg.at[pl.ds(base, ROWS_LAST), :],
            out_hbm.at[cid, pl.ds(base, ROWS_LAST), :],
        )


@functools.cache
def _make_sc_edge():
    return pl.kernel(
        _sc_edge_body,
        out_type=jax.ShapeDtypeStruct((NC, N, D_OUT), jnp.float32),
        mesh=plsc.VectorSubcoreMesh(
            core_axis_name="c", subcore_axis_name="s", num_cores=NC, num_subcores=NS
        ),
        scratch_types=[
            pltpu.VMEM((CH,), jnp.int32),
            pltpu.VMEM((CH,), jnp.int32),
            pltpu.VMEM((CH, D_OUT), jnp.float32),
            pltpu.VMEM((CH, D_OUT), jnp.float32),
            pltpu.VMEM((CH + LANES,), jnp.float32),
            pltpu.VMEM((ZR, D_OUT), jnp.float32),
            pltpu.VMEM_SHARED((N, D_OUT), jnp.float32),
            pltpu.SemaphoreType.DMA,
        ],
    )


@jax.jit
def kernel(node_feats, edge_feats, degs, norm, Wn, bn, We, be, res_w, edge_index):
    src = edge_index[0]
    dst = edge_index[1]
    norm_flat = norm.reshape(E)
    bn2 = bn.reshape(1, D_OUT)
    be2 = be.reshape(1, D_OUT)

    h = pl.pallas_call(
        _h_body,
        grid=(N // BN,),
        in_specs=[
            pl.BlockSpec((BN, D_IN), lambda i: (i, 0)),
            pl.BlockSpec((D_OUT, D_IN), lambda i: (0, 0)),
            pl.BlockSpec((1, D_OUT), lambda i: (0, 0)),
        ],
        out_specs=pl.BlockSpec((BN, D_OUT), lambda i: (i, 0)),
        out_shape=jax.ShapeDtypeStruct((N, D_OUT), jnp.float32),
    )(node_feats, Wn, bn2)

    ef = pl.pallas_call(
        _ef_body,
        grid=(E // BE,),
        in_specs=[
            pl.BlockSpec((BE, D_EDGE), lambda i: (i, 0)),
            pl.BlockSpec((D_OUT, D_EDGE), lambda i: (0, 0)),
            pl.BlockSpec((1, D_OUT), lambda i: (0, 0)),
        ],
        out_specs=pl.BlockSpec((BE, D_OUT), lambda i: (i, 0)),
        out_shape=jax.ShapeDtypeStruct((E, D_OUT), jnp.float32),
    )(edge_feats, We, be2)

    partials = _make_sc_edge()(src, dst, norm_flat, ef, h)

    out = pl.pallas_call(
        _combine_body,
        grid=(N // BN,),
        in_specs=[
            pl.BlockSpec((NC, BN, D_OUT), lambda i: (0, i, 0)),
            pl.BlockSpec((BN, D_OUT), lambda i: (i, 0)),
            pl.BlockSpec((1, D_OUT), lambda i: (0, 0)),
            pl.BlockSpec((BN, 1), lambda i: (i, 0)),
        ],
        out_specs=pl.BlockSpec((BN, D_OUT), lambda i: (i, 0)),
        out_shape=jax.ShapeDtypeStruct((N, D_OUT), jnp.float32),
    )(partials, h, res_w, degs)

    return out


# speedup vs baseline: 3.6653x; 1.5191x over previous
import functools

import jax
import jax.numpy as jnp
from jax import lax
from jax.experimental import pallas as pl
from jax.experimental.pallas import tpu as pltpu
from jax.experimental.pallas import tpu_sc as plsc

N = 10000
E = 320000
D_IN = 128
D_EDGE = 16
D_OUT = 128

NC = 2
NS = 16
NW = NC * NS
LANES = 16

EW = E // NW
CH = 40
NCHUNK = EW // CH
ROWS_MAIN = 640
ROWS_LAST = N - (NS - 1) * ROWS_MAIN
ZR = 40
BN = 1000
BE = 4000


def _h_body(nf_ref, wn_ref, bn_ref, h_ref):
    h_ref[...] = (
        lax.dot_general(
            nf_ref[...], wn_ref[...],
            (((1,), (1,)), ((), ())),
            preferred_element_type=jnp.float32,
        )
        + bn_ref[...]
    )


def _ef_body(ef_in_ref, we_ref, be_ref, ef_ref):
    ef_ref[...] = (
        lax.dot_general(
            ef_in_ref[...], we_ref[...],
            (((1,), (1,)), ((), ())),
            preferred_element_type=jnp.float32,
        )
        + be_ref[...]
    )


def _combine_body(p_ref, h_ref, resw_ref, degs_ref, o_ref):
    h = h_ref[...]
    res = jnp.maximum(h + resw_ref[...], 0.0) * (1.0 / degs_ref[...])
    o_ref[...] = p_ref[0] + p_ref[1] + res


def _sc_edge_body(src_hbm, dst_hbm, norm_hbm, ef_hbm, h_hbm, out_hbm,
                  srcb0, srcb1, dstb0, dstb1, normb0, normb1,
                  rows0, rows1, ef0, ef1, zbuf, agg,
                  sem_m0, sem_m1, sem_g0, sem_g1, sem_e0, sem_e1):
    cid = lax.axis_index("c")
    sid = lax.axis_index("s")
    wid = cid * NS + sid

    srcb = (srcb0, srcb1)
    dstb = (dstb0, dstb1)
    normb = (normb0, normb1)
    rows = (rows0, rows1)
    efb = (ef0, ef1)
    sem_m = (sem_m0, sem_m1)
    sem_g = (sem_g0, sem_g1)
    sem_e = (sem_e0, sem_e1)

    ebase0 = wid * EW

    def _meta_load(ci, b):
        off = ebase0 + ci * CH
        pltpu.async_copy(src_hbm.at[pl.ds(off, CH)], srcb[b], sem_m[b])
        pltpu.async_copy(dst_hbm.at[pl.ds(off, CH)], dstb[b], sem_m[b])
        pltpu.async_copy(
            norm_hbm.at[pl.ds(off, CH)], normb[b].at[pl.ds(0, CH)], sem_m[b]
        )

    def _meta_wait(ci, b):
        off = ebase0 + ci * CH
        pltpu.make_async_copy(
            src_hbm.at[pl.ds(off, CH)], srcb[b], sem_m[b]
        ).wait()
        pltpu.make_async_copy(
            dst_hbm.at[pl.ds(off, CH)], dstb[b], sem_m[b]
        ).wait()
        pltpu.make_async_copy(
            norm_hbm.at[pl.ds(off, CH)], normb[b].at[pl.ds(0, CH)], sem_m[b]
        ).wait()

    _meta_load(0, 0)
    _meta_load(1, 1)

    def _zero_row(r, carry):
        for c8 in range(D_OUT // LANES):
            zbuf[r, pl.ds(c8 * LANES, LANES)] = jnp.zeros((LANES,), jnp.float32)
        return carry

    lax.fori_loop(0, ZR, _zero_row, 0)
    r0 = pl.multiple_of(sid * ROWS_MAIN, ROWS_MAIN)

    @pl.when(sid < NS - 1)
    def _zero_main():
        for k in range(ROWS_MAIN // ZR):
            pltpu.sync_copy(zbuf, agg.at[pl.ds(r0 + k * ZR, ZR), :])

    @pl.when(sid == NS - 1)
    def _zero_last():
        base = (NS - 1) * ROWS_MAIN
        for k in range(ROWS_LAST // ZR):
            pltpu.sync_copy(zbuf, agg.at[pl.ds(base + k * ZR, ZR), :])

    plsc.subcore_barrier()

    ebase = wid * EW

    def _load(ci, b):
        pltpu.async_copy(h_hbm.at[srcb[b]], rows[b], sem_g[b])
        pltpu.async_copy(
            ef_hbm.at[pl.ds(ebase + ci * CH, CH), :], efb[b], sem_e[b]
        )

    def _wait_loads(ci, b):
        pltpu.make_async_copy(h_hbm.at[srcb[b]], rows[b], sem_g[b]).wait()
        pltpu.make_async_copy(
            ef_hbm.at[pl.ds(ebase + ci * CH, CH), :], efb[b], sem_e[b]
        ).wait()

    def _body(ci, b, n_ahead):
        _wait_loads(ci, b)
        if n_ahead >= 1:
            _meta_wait(ci + 1, 1 - b)
            _load(ci + 1, 1 - b)
        rv = rows[b]
        ev = efb[b]
        nb = normb[b]

        def _edge(e, c2):
            nv = nb[pl.ds(e, LANES)]
            nsp = lax.broadcast(nv[0], (LANES,))
            for v in range(D_OUT // LANES):
                x = rv[e, pl.ds(v * LANES, LANES)]
                y = ev[e, pl.ds(v * LANES, LANES)]
                rv[e, pl.ds(v * LANES, LANES)] = jnp.maximum(x + y, 0.0) * nsp
            return c2

        lax.fori_loop(0, CH, _edge, 0)
        pltpu.sync_copy(rv, agg.at[dstb[b]], add=True)
        if n_ahead >= 2:
            _meta_load(ci + 2, b)

    _meta_wait(0, 0)
    _load(0, 0)

    def _group(j, carry):
        ci = 2 * j
        _body(ci, 0, 2)
        _body(ci + 1, 1, 2)
        return carry

    lax.fori_loop(0, (NCHUNK - 2) // 2, _group, 0)
    _body(NCHUNK - 2, 0, 1)
    _body(NCHUNK - 1, 1, 0)
    plsc.subcore_barrier()

    @pl.when(sid < NS - 1)
    def _pub_main():
        pltpu.sync_copy(
            agg.at[pl.ds(r0, ROWS_MAIN), :],
            out_hbm.at[cid, pl.ds(r0, ROWS_MAIN), :],
        )

    @pl.when(sid == NS - 1)
    def _pub_last():
        base = (NS - 1) * ROWS_MAIN
        pltpu.sync_copy(
            agg.at[pl.ds(base, ROWS_LAST), :],
            out_hbm.at[cid, pl.ds(base, ROWS_LAST), :],
        )


@functools.cache
def _make_sc_edge():
    return pl.kernel(
        _sc_edge_body,
        out_type=jax.ShapeDtypeStruct((NC, N, D_OUT), jnp.float32),
        mesh=plsc.VectorSubcoreMesh(
            core_axis_name="c", subcore_axis_name="s", num_cores=NC, num_subcores=NS
        ),
        scratch_types=[
            pltpu.VMEM((CH,), jnp.int32),
            pltpu.VMEM((CH,), jnp.int32),
            pltpu.VMEM((CH,), jnp.int32),
            pltpu.VMEM((CH,), jnp.int32),
            pltpu.VMEM((CH + LANES,), jnp.float32),
            pltpu.VMEM((CH + LANES,), jnp.float32),
            pltpu.VMEM((CH, D_OUT), jnp.float32),
            pltpu.VMEM((CH, D_OUT), jnp.float32),
            pltpu.VMEM((CH, D_OUT), jnp.float32),
            pltpu.VMEM((CH, D_OUT), jnp.float32),
            pltpu.VMEM((ZR, D_OUT), jnp.float32),
            pltpu.VMEM_SHARED((N, D_OUT), jnp.float32),
            pltpu.SemaphoreType.DMA,
            pltpu.SemaphoreType.DMA,
            pltpu.SemaphoreType.DMA,
            pltpu.SemaphoreType.DMA,
            pltpu.SemaphoreType.DMA,
            pltpu.SemaphoreType.DMA,
        ],
    )


@jax.jit
def kernel(node_feats, edge_feats, degs, norm, Wn, bn, We, be, res_w, edge_index):
    src = edge_index[0]
    dst = edge_index[1]
    norm_flat = norm.reshape(E)
    bn2 = bn.reshape(1, D_OUT)
    be2 = be.reshape(1, D_OUT)

    h = pl.pallas_call(
        _h_body,
        grid=(N // BN,),
        in_specs=[
            pl.BlockSpec((BN, D_IN), lambda i: (i, 0)),
            pl.BlockSpec((D_OUT, D_IN), lambda i: (0, 0)),
            pl.BlockSpec((1, D_OUT), lambda i: (0, 0)),
        ],
        out_specs=pl.BlockSpec((BN, D_OUT), lambda i: (i, 0)),
        out_shape=jax.ShapeDtypeStruct((N, D_OUT), jnp.float32),
    )(node_feats, Wn, bn2)

    ef = pl.pallas_call(
        _ef_body,
        grid=(E // BE,),
        in_specs=[
            pl.BlockSpec((BE, D_EDGE), lambda i: (i, 0)),
            pl.BlockSpec((D_OUT, D_EDGE), lambda i: (0, 0)),
            pl.BlockSpec((1, D_OUT), lambda i: (0, 0)),
        ],
        out_specs=pl.BlockSpec((BE, D_OUT), lambda i: (i, 0)),
        out_shape=jax.ShapeDtypeStruct((E, D_OUT), jnp.float32),
    )(edge_feats, We, be2)

    partials = _make_sc_edge()(src, dst, norm_flat, ef, h)

    out = pl.pallas_call(
        _combine_body,
        grid=(N // BN,),
        in_specs=[
            pl.BlockSpec((NC, BN, D_OUT), lambda i: (0, i, 0)),
            pl.BlockSpec((BN, D_OUT), lambda i: (i, 0)),
            pl.BlockSpec((1, D_OUT), lambda i: (0, 0)),
            pl.BlockSpec((BN, 1), lambda i: (i, 0)),
        ],
        out_specs=pl.BlockSpec((BN, D_OUT), lambda i: (i, 0)),
        out_shape=jax.ShapeDtypeStruct((N, D_OUT), jnp.float32),
    )(partials, h, res_w, degs)

    return out
